# baseline (speedup 1.0000x reference)
import jax
import jax.numpy as jnp
import numpy as np
from jax.experimental import pallas as pl

_N = 197
_C = 384
_H = 6
_DH = 64
_GS = 49
_G = 4
_TOPK = 2


def _attn_kernel(x_ref, wqkv_ref, bqkv_ref, wproj_ref, bproj_ref, e_ref, o_ref):
    xb = x_ref[0]
    qkv = (
        jnp.dot(xb, wqkv_ref[...], preferred_element_type=jnp.float32)
        + bqkv_ref[...]
    )
    scale = _DH ** -0.5

    k_all = qkv[:, _C:2 * _C]
    gmaxs = []
    gmins = []
    for g in range(_G):
        kg = k_all[1 + g * _GS:1 + (g + 1) * _GS, :]
        gmaxs.append(jnp.max(kg, axis=0, keepdims=True))
        gmins.append(jnp.min(kg, axis=0, keepdims=True))
    gsum = jnp.concatenate([a + b for a, b in zip(gmaxs, gmins)], axis=0)
    gdif = jnp.concatenate([a - b for a, b in zip(gmaxs, gmins)], axis=0)

    row1 = jax.lax.broadcasted_iota(jnp.int32, (_N, 1), 0)
    ones_col = jnp.ones((_N, 1), jnp.float32)

    outs = []
    for h in range(_H):
        lo = h * _DH
        hi = lo + _DH
        q = qkv[:, lo:hi] * scale
        k = qkv[:, _C + lo:_C + hi]
        v = qkv[:, 2 * _C + lo:2 * _C + hi]

        q2 = jnp.concatenate([q, jnp.abs(q)], axis=1)
        m2 = jnp.concatenate([gsum[:, lo:hi], gdif[:, lo:hi]], axis=1)
        scores = jax.lax.dot_general(
            q2, m2, (((1,), (1,)), ((), ())), preferred_element_type=jnp.float32
        )

        scols = [scores[:, g:g + 1] for g in range(_G)]
        sel = []
        for g in range(_G):
            rank = jnp.zeros((_N, 1), jnp.float32)
            for j in range(_G):
                if j == g:
                    continue
                cmp = (scols[j] > scols[g]).astype(jnp.float32)
                if j < g:
                    cmp = cmp + (scols[j] == scols[g]).astype(jnp.float32)
                rank = rank + cmp
            sel.append((rank < _TOPK).astype(jnp.float32))

        s_mat = jnp.concatenate(sel + [ones_col], axis=1)
        s_mat = jnp.where(row1 == 0, 1.0, s_mat)
        km = jnp.dot(s_mat, e_ref[...], preferred_element_type=jnp.float32)

        logits = jax.lax.dot_general(
            q, k, (((1,), (1,)), ((), ())), preferred_element_type=jnp.float32
        )
        logits = logits * km
        m = jnp.max(logits, axis=1, keepdims=True)
        e = jnp.exp(logits - m)
        s = jnp.sum(e, axis=1, keepdims=True)
        ov = jnp.dot(e, v, preferred_element_type=jnp.float32)
        outs.append(ov * (1.0 / s))

    out = jnp.concatenate(outs, axis=1)
    o_ref[0] = (
        jnp.dot(out, wproj_ref[...], preferred_element_type=jnp.float32)
        + bproj_ref[...]
    )


def _make_emat():
    e = np.zeros((_G + 1, _N), np.float32)
    for g in range(_G):
        e[g, 1 + g * _GS:1 + (g + 1) * _GS] = 1.0
    e[_G, 0] = 1.0
    return jnp.asarray(e)


def kernel(x, Wqkv, bqkv, Wproj, bproj):
    Bsz = x.shape[0]
    wqkv_t = Wqkv.T
    wproj_t = Wproj.T
    bqkv2 = bqkv.reshape(1, -1)
    bproj2 = bproj.reshape(1, -1)
    emat = _make_emat()
    return pl.pallas_call(
        _attn_kernel,
        grid=(Bsz,),
        in_specs=[
            pl.BlockSpec((1, _N, _C), lambda b: (b, 0, 0)),
            pl.BlockSpec((_C, 3 * _C), lambda b: (0, 0)),
            pl.BlockSpec((1, 3 * _C), lambda b: (0, 0)),
            pl.BlockSpec((_C, _C), lambda b: (0, 0)),
            pl.BlockSpec((1, _C), lambda b: (0, 0)),
            pl.BlockSpec((_G + 1, _N), lambda b: (0, 0)),
        ],
        out_specs=pl.BlockSpec((1, _N, _C), lambda b: (b, 0, 0)),
        out_shape=jax.ShapeDtypeStruct(x.shape, x.dtype),
    )(x, wqkv_t, bqkv2, wproj_t, bproj2, emat)


# speedup vs baseline: 1.4555x; 1.4555x over previous
import jax
import jax.numpy as jnp
from jax.experimental import pallas as pl

_N = 197
_C = 384
_H = 6
_DH = 64
_GS = 49
_G = 4
_TOPK = 2


def _attn_kernel(x_ref, wqkv_ref, bqkv_ref, wproj_ref, bproj_ref, o_ref):
    xb = x_ref[0]
    qkv = (
        jax.lax.dot_general(
            xb, wqkv_ref[...], (((1,), (1,)), ((), ())),
            preferred_element_type=jnp.float32,
        )
        + bqkv_ref[...]
    )
    scale = _DH ** -0.5

    col = jax.lax.broadcasted_iota(jnp.int32, (_N, _N), 1)
    grpcol = (col - 1) // _GS
    gmasks = [(grpcol == g).astype(jnp.float32) for g in range(_G)]
    cls_col = (col == 0).astype(jnp.float32)
    row1 = jax.lax.broadcasted_iota(jnp.int32, (_N, 1), 0)

    k_all = qkv[:, _C:2 * _C]
    gstats = []
    for g in range(_G):
        kg = k_all[1 + g * _GS:1 + (g + 1) * _GS, :]
        gstats.append((jnp.max(kg, axis=0, keepdims=True),
                       jnp.min(kg, axis=0, keepdims=True)))

    outs = []
    for h in range(_H):
        lo = h * _DH
        hi = lo + _DH
        q = qkv[:, lo:hi]
        k = qkv[:, _C + lo:_C + hi]
        v = qkv[:, 2 * _C + lo:2 * _C + hi]

        qpos = q >= 0.0
        scols = []
        for g in range(_G):
            gmax, gmin = gstats[g]
            ms = jnp.where(qpos, gmax[:, lo:hi], gmin[:, lo:hi])
            scols.append(jnp.sum(q * ms, axis=1, keepdims=True))

        sel = []
        for g in range(_G):
            rank = jnp.zeros((_N, 1), jnp.float32)
            for j in range(_G):
                if j == g:
                    continue
                cmp = (scols[j] > scols[g]).astype(jnp.float32)
                if j < g:
                    cmp = cmp + (scols[j] == scols[g]).astype(jnp.float32)
                rank = rank + cmp
            keep = jnp.logical_or(rank < _TOPK, row1 == 0)
            sel.append(keep.astype(jnp.float32))

        km = cls_col + sel[0] * gmasks[0]
        for g in range(1, _G):
            km = km + sel[g] * gmasks[g]

        logits = jax.lax.dot_general(
            q, k, (((1,), (1,)), ((), ())), preferred_element_type=jnp.float32
        )
        logits = logits * km * scale
        m = jnp.max(logits, axis=1, keepdims=True)
        e = jnp.exp(logits - m)
        s = jnp.sum(e, axis=1, keepdims=True)
        ov = jnp.dot(e, v, preferred_element_type=jnp.float32)
        outs.append(ov * (1.0 / s))

    out = jnp.concatenate(outs, axis=1)
    o_ref[0] = (
        jax.lax.dot_general(
            out, wproj_ref[...], (((1,), (1,)), ((), ())),
            preferred_element_type=jnp.float32,
        )
        + bproj_ref[...]
    )


def kernel(x, Wqkv, bqkv, Wproj, bproj):
    Bsz = x.shape[0]
    bqkv2 = bqkv.reshape(1, -1)
    bproj2 = bproj.reshape(1, -1)
    return pl.pallas_call(
        _attn_kernel,
        grid=(Bsz,),
        in_specs=[
            pl.BlockSpec((1, _N, _C), lambda b: (b, 0, 0)),
            pl.BlockSpec((3 * _C, _C), lambda b: (0, 0)),
            pl.BlockSpec((1, 3 * _C), lambda b: (0, 0)),
            pl.BlockSpec((_C, _C), lambda b: (0, 0)),
            pl.BlockSpec((1, _C), lambda b: (0, 0)),
        ],
        out_specs=pl.BlockSpec((1, _N, _C), lambda b: (b, 0, 0)),
        out_shape=jax.ShapeDtypeStruct(x.shape, x.dtype),
    )(x, Wqkv, bqkv2, Wproj, bproj2)


# speedup vs baseline: 1.5292x; 1.0506x over previous
import jax
import jax.numpy as jnp
from jax.experimental import pallas as pl
from jax.experimental.pallas import tpu as pltpu

_N = 197
_C = 384
_H = 6
_DH = 64
_GS = 49
_G = 4
_TOPK = 2


def _attn_kernel(x_ref, wqkv_ref, bqkv_ref, wproj_ref, bproj_ref, o_ref):
    xb = x_ref[0]
    qkv = (
        jax.lax.dot_general(
            xb, wqkv_ref[...], (((1,), (1,)), ((), ())),
            preferred_element_type=jnp.float32,
        )
        + bqkv_ref[...]
    )
    scale = _DH ** -0.5

    col = jax.lax.broadcasted_iota(jnp.int32, (_N, _N), 1)
    grpcol = (col - 1) // _GS
    gmasks = [(grpcol == g).astype(jnp.float32) for g in range(_G)]
    cls_col = (col == 0).astype(jnp.float32)
    row1 = jax.lax.broadcasted_iota(jnp.int32, (_N, 1), 0)
    rank0 = jnp.where(row1 == 0, -float(_G), 0.0)

    k_all = qkv[:, _C:2 * _C]
    gstats = []
    for g in range(_G):
        kg = k_all[1 + g * _GS:1 + (g + 1) * _GS, :]
        gstats.append((jnp.max(kg, axis=0, keepdims=True),
                       jnp.min(kg, axis=0, keepdims=True)))

    outs = []
    for h in range(_H):
        lo = h * _DH
        hi = lo + _DH
        q = qkv[:, lo:hi]
        k = qkv[:, _C + lo:_C + hi]
        v = qkv[:, 2 * _C + lo:2 * _C + hi]

        qpos = q >= 0.0
        scols = []
        for g in range(_G):
            gmax, gmin = gstats[g]
            ms = jnp.where(qpos, gmax[:, lo:hi], gmin[:, lo:hi])
            scols.append(jnp.sum(q * ms, axis=1, keepdims=True))

        a = {}
        for j in range(_G):
            for g in range(j + 1, _G):
                a[(j, g)] = (scols[j] >= scols[g]).astype(jnp.float32)
        sel = []
        for g in range(_G):
            rank = rank0 + float(_G - 1 - g)
            for j in range(g):
                rank = rank + a[(j, g)]
            for j in range(g + 1, _G):
                rank = rank - a[(g, j)]
            sel.append((rank < _TOPK).astype(jnp.float32))

        km = cls_col + sel[0] * gmasks[0]
        for g in range(1, _G):
            km = km + sel[g] * gmasks[g]

        logits = jax.lax.dot_general(
            q * scale, k, (((1,), (1,)), ((), ())),
            preferred_element_type=jnp.float32,
        )
        logits = logits * km
        m = jnp.max(logits, axis=1, keepdims=True)
        e = jnp.exp(logits - m)
        s = jnp.sum(e, axis=1, keepdims=True)
        ov = jnp.dot(e, v, preferred_element_type=jnp.float32)
        outs.append(ov * (1.0 / s))

    out = jnp.concatenate(outs, axis=1)
    o_ref[0] = (
        jax.lax.dot_general(
            out, wproj_ref[...], (((1,), (1,)), ((), ())),
            preferred_element_type=jnp.float32,
        )
        + bproj_ref[...]
    )


def kernel(x, Wqkv, bqkv, Wproj, bproj):
    Bsz = x.shape[0]
    bqkv2 = bqkv.reshape(1, -1)
    bproj2 = bproj.reshape(1, -1)
    return pl.pallas_call(
        _attn_kernel,
        grid=(Bsz,),
        in_specs=[
            pl.BlockSpec((1, _N, _C), lambda b: (b, 0, 0)),
            pl.BlockSpec((3 * _C, _C), lambda b: (0, 0)),
            pl.BlockSpec((1, 3 * _C), lambda b: (0, 0)),
            pl.BlockSpec((_C, _C), lambda b: (0, 0)),
            pl.BlockSpec((1, _C), lambda b: (0, 0)),
        ],
        out_specs=pl.BlockSpec((1, _N, _C), lambda b: (b, 0, 0)),
        out_shape=jax.ShapeDtypeStruct(x.shape, x.dtype),
        compiler_params=pltpu.CompilerParams(
            dimension_semantics=("parallel",),
        ),
    )(x, Wqkv, bqkv2, Wproj, bproj2)
